# baseline (speedup 1.0000x reference)
import functools

import jax
import jax.numpy as jnp
from jax import lax
from jax.experimental import pallas as pl
from jax.experimental.pallas import tpu as pltpu
from jax.experimental.pallas import tpu_sc as plsc

N = 10000
E = 320000
D = 128
NC = 2
NS = 16
NW = NC * NS
K = 80
NCH2 = 252
EPS = NCH2 * K
E_PAD = NS * EPS
NPAD = 10240
HALF = NPAD // NC
TRASH = HALF
RPT = HALF // NS
RB = 32

_mesh = plsc.VectorSubcoreMesh(core_axis_name="c", subcore_axis_name="s")


def _deg_body(col_hbm, ones_hbm, zeros_hbm, out_hbm,
              col_v, ones_v, blk_v, acc, sem):
    c = lax.axis_index("c")
    s = lax.axis_index("s")
    pltpu.sync_copy(col_hbm.at[s], col_v)
    pltpu.sync_copy(ones_hbm, ones_v)

    base = jnp.broadcast_to((c * HALF).astype(jnp.int32), (16,))
    trash = jnp.broadcast_to(jnp.int32(TRASH), (16,))

    def remap(j, _):
        for m in range(K // 16):
            v = col_v[j, pl.ds(m * 16, 16)]
            t = v - base
            ok = (t >= 0) & (t < HALF)
            col_v[j, pl.ds(m * 16, 16)] = jnp.where(ok, t, trash)
        return 0
    lax.fori_loop(0, NCH2, remap, 0)

    pltpu.sync_copy(zeros_hbm, blk_v)
    for k in range(RPT // RB):
        pltpu.sync_copy(blk_v, acc.at[pl.ds(s * RPT + k * RB, RB)])
    plsc.subcore_barrier()

    def sstart(j):
        pltpu.async_copy(ones_v, acc.at[col_v.at[j]], sem, add=True)

    def swait():
        pltpu.make_async_copy(ones_v, acc.at[col_v.at[0]], sem).wait()

    for j in range(4):
        sstart(j)

    def chunk(j, _):
        swait()
        sstart(j + 4)
        return 0
    lax.fori_loop(0, NCH2 - 4, chunk, 0)
    for _ in range(4):
        swait()

    plsc.subcore_barrier()
    for k in range(RPT // RB):
        r0 = s * RPT + k * RB
        pltpu.sync_copy(acc.at[pl.ds(r0, RB)], blk_v)
        pltpu.sync_copy(blk_v, out_hbm.at[pl.ds(c * HALF + r0, RB)])


@functools.partial(
    pl.kernel,
    out_type=jax.ShapeDtypeStruct((NPAD, D), jnp.float32),
    mesh=_mesh,
    scratch_types=[
        pltpu.VMEM((NCH2, K), jnp.int32),
        pltpu.VMEM((K, D), jnp.float32),
        pltpu.VMEM((RB, D), jnp.float32),
        pltpu.MemorySpace.VMEM_SHARED((HALF + 8, D), jnp.float32),
        pltpu.SemaphoreType.DMA,
    ],
)
def _deg_kernel(*refs):
    _deg_body(*refs)


def _scatter_body(h_hbm, row_hbm, col_hbm, zeros_hbm, out_hbm,
                  row_v, col_v, rows_a, rows_b, rows_c, blk_v, acc,
                  gsem_a, gsem_b, ssem_a, ssem_b, gsem_c, ssem_c):
    c = lax.axis_index("c")
    s = lax.axis_index("s")
    pltpu.sync_copy(row_hbm.at[pl.ds(s * EPS, EPS)], row_v)
    pltpu.sync_copy(col_hbm.at[s], col_v)

    base = jnp.broadcast_to((c * HALF).astype(jnp.int32), (16,))
    trash = jnp.broadcast_to(jnp.int32(TRASH), (16,))

    def remap(j, _):
        for m in range(K // 16):
            v = col_v[j, pl.ds(m * 16, 16)]
            t = v - base
            ok = (t >= 0) & (t < HALF)
            col_v[j, pl.ds(m * 16, 16)] = jnp.where(ok, t, trash)
        return 0
    lax.fori_loop(0, NCH2, remap, 0)

    pltpu.sync_copy(zeros_hbm, blk_v)
    for k in range(RPT // RB):
        pltpu.sync_copy(blk_v, acc.at[pl.ds(s * RPT + k * RB, RB)])
    plsc.subcore_barrier()

    def gather(j, buf, sem):
        pltpu.async_copy(h_hbm.at[row_v.at[pl.ds(j * K, K)]], buf, sem)

    def gwait(buf, sem):
        pltpu.make_async_copy(h_hbm.at[row_v.at[pl.ds(0, K)]], buf, sem).wait()

    def sstart(j, buf, sem):
        pltpu.async_copy(buf, acc.at[col_v.at[j]], sem, add=True)

    def swait(buf, sem):
        pltpu.make_async_copy(buf, acc.at[col_v.at[0]], sem).wait()

    bufs = ((rows_a, gsem_a, ssem_a), (rows_b, gsem_b, ssem_b),
            (rows_c, gsem_c, ssem_c))
    for j in range(3):
        gather(j, bufs[j][0], bufs[j][1])

    def step(i, _):
        j = i * 3
        for t in range(3):
            buf, gs, ss = bufs[t]
            gwait(buf, gs)
            sstart(j + t, buf, ss)
        for t in range(3):
            buf, gs, ss = bufs[t]
            swait(buf, ss)
            gather(j + 3 + t, buf, gs)
        return 0

    lax.fori_loop(0, NCH2 // 3 - 1, step, 0)
    j = NCH2 - 3
    for t in range(3):
        buf, gs, ss = bufs[t]
        gwait(buf, gs)
        sstart(j + t, buf, ss)
    for t in range(3):
        swait(bufs[t][0], bufs[t][2])

    plsc.subcore_barrier()
    for k in range(RPT // RB):
        r0 = s * RPT + k * RB
        pltpu.sync_copy(acc.at[pl.ds(r0, RB)], blk_v)
        pltpu.sync_copy(blk_v, out_hbm.at[pl.ds(c * HALF + r0, RB)])


@functools.partial(
    pl.kernel,
    out_type=jax.ShapeDtypeStruct((NPAD, D), jnp.float32),
    mesh=_mesh,
    scratch_types=[
        pltpu.VMEM((EPS,), jnp.int32),
        pltpu.VMEM((NCH2, K), jnp.int32),
        pltpu.VMEM((K, D), jnp.float32),
        pltpu.VMEM((K, D), jnp.float32),
        pltpu.VMEM((K, D), jnp.float32),
        pltpu.VMEM((RB, D), jnp.float32),
        pltpu.MemorySpace.VMEM_SHARED((HALF + 8, D), jnp.float32),
        pltpu.SemaphoreType.DMA,
        pltpu.SemaphoreType.DMA,
        pltpu.SemaphoreType.DMA,
        pltpu.SemaphoreType.DMA,
        pltpu.SemaphoreType.DMA,
        pltpu.SemaphoreType.DMA,
    ],
)
def _edge_scatter(*refs):
    _scatter_body(*refs)


BS = 2000


def _mm_scale_body(x_ref, w_ref, b_ref, dis_ref, o_ref):
    acc = lax.dot_general(x_ref[...], w_ref[...], (((1,), (1,)), ((), ())),
                          preferred_element_type=jnp.float32)
    o_ref[...] = (acc + b_ref[...]) * dis_ref[...]


def _mm1(x, W, b2d, dis):
    return pl.pallas_call(
        _mm_scale_body,
        grid=(N // BS,),
        in_specs=[
            pl.BlockSpec((BS, D), lambda i: (i, 0)),
            pl.BlockSpec((D, D), lambda i: (0, 0)),
            pl.BlockSpec((1, D), lambda i: (0, 0)),
            pl.BlockSpec((BS, 1), lambda i: (i, 0)),
        ],
        out_specs=pl.BlockSpec((BS, D), lambda i: (i, 0)),
        out_shape=jax.ShapeDtypeStruct((N, D), jnp.float32),
    )(x, W, b2d, dis)


def _mid_body(sp_ref, h_ref, w_ref, b_ref, dis_ref, o_ref):
    u = jnp.maximum((sp_ref[...] + h_ref[...]) * dis_ref[...], 0.0)
    acc = lax.dot_general(u, w_ref[...], (((1,), (1,)), ((), ())),
                          preferred_element_type=jnp.float32)
    o_ref[...] = (acc + b_ref[...]) * dis_ref[...]


def _mm2(sp, h, W, b2d, dis):
    return pl.pallas_call(
        _mid_body,
        grid=(N // BS,),
        in_specs=[
            pl.BlockSpec((BS, D), lambda i: (i, 0)),
            pl.BlockSpec((BS, D), lambda i: (i, 0)),
            pl.BlockSpec((D, D), lambda i: (0, 0)),
            pl.BlockSpec((1, D), lambda i: (0, 0)),
            pl.BlockSpec((BS, 1), lambda i: (i, 0)),
        ],
        out_specs=pl.BlockSpec((BS, D), lambda i: (i, 0)),
        out_shape=jax.ShapeDtypeStruct((N, D), jnp.float32),
    )(sp, h, W, b2d, dis)


def _final_body(sp_ref, h_ref, dis_ref, o_ref):
    o_ref[...] = jnp.maximum((sp_ref[...] + h_ref[...]) * dis_ref[...], 0.0)


def _mm3(sp, h, dis):
    return pl.pallas_call(
        _final_body,
        grid=(N // BS,),
        in_specs=[
            pl.BlockSpec((BS, D), lambda i: (i, 0)),
            pl.BlockSpec((BS, D), lambda i: (i, 0)),
            pl.BlockSpec((BS, 1), lambda i: (i, 0)),
        ],
        out_specs=pl.BlockSpec((BS, D), lambda i: (i, 0)),
        out_shape=jax.ShapeDtypeStruct((N, D), jnp.float32),
    )(sp, h, dis)


def kernel(x, edge_index, W1, b1, W2, b2):
    x2 = x[0]
    row = edge_index[0].astype(jnp.int32)
    col = edge_index[1].astype(jnp.int32)
    pad = E_PAD - E
    rowg = jnp.concatenate([row, jnp.zeros((pad,), jnp.int32)])
    rowd = jnp.concatenate([row, jnp.full((pad,), N, jnp.int32)])
    colp = jnp.concatenate([col, jnp.full((pad,), N, jnp.int32)])
    rowd_s = rowd.reshape(NS, NCH2, K)
    col_s = colp.reshape(NS, NCH2, K)
    zeros128 = jnp.zeros((RB, D), jnp.float32)
    ones128 = jnp.ones((K, D), jnp.float32)
    b1_2d = b1.reshape(1, D)
    b2_2d = b2.reshape(1, D)

    degw = _deg_kernel(rowd_s, ones128, zeros128)
    deg = degw[:N, 0] + 1.0
    dis = lax.rsqrt(deg)[:, None]

    h1 = _mm1(x2, W1, b1_2d, dis)
    s1 = _edge_scatter(h1, rowg, col_s, zeros128)
    h2 = _mm2(s1, h1, W2, b2_2d, dis)
    s2 = _edge_scatter(h2, rowg, col_s, zeros128)
    out = _mm3(s2, h2, dis)
    return out[None]


# speedup vs baseline: 1.3270x; 1.3270x over previous
import functools

import jax
import jax.numpy as jnp
from jax import lax
from jax.experimental import pallas as pl
from jax.experimental.pallas import tpu as pltpu
from jax.experimental.pallas import tpu_sc as plsc

N = 10000
E = 320000
D = 128
NC = 2
NS = 16
NW = NC * NS
K = 80
NCH2 = 250
EPS = NCH2 * K
E_PAD = NS * EPS
NPAD = 10240
HALF = NPAD // NC
TRASH = HALF
RPT = HALF // NS
RB = 32

_mesh = plsc.VectorSubcoreMesh(core_axis_name="c", subcore_axis_name="s")


def _deg_body(col_hbm, ones_hbm, zeros_hbm, out_hbm,
              col_v, ones_v, blk_v, acc, sem):
    c = lax.axis_index("c")
    s = lax.axis_index("s")
    pltpu.sync_copy(col_hbm.at[s], col_v)
    pltpu.sync_copy(ones_hbm, ones_v)

    base = jnp.broadcast_to((c * HALF).astype(jnp.int32), (16,))
    trash = jnp.broadcast_to(jnp.int32(TRASH), (16,))

    def remap(j, _):
        for m in range(K // 16):
            v = col_v[j, pl.ds(m * 16, 16)]
            t = v - base
            ok = (t >= 0) & (t < HALF)
            col_v[j, pl.ds(m * 16, 16)] = jnp.where(ok, t, trash)
        return 0
    lax.fori_loop(0, NCH2, remap, 0)

    pltpu.sync_copy(zeros_hbm, blk_v)
    for k in range(RPT // RB):
        pltpu.sync_copy(blk_v, acc.at[pl.ds(s * RPT + k * RB, RB)])
    plsc.subcore_barrier()

    def chunk(j, _):
        pltpu.sync_copy(ones_v, acc.at[col_v.at[j]], add=True)
        return 0
    lax.fori_loop(0, NCH2, chunk, 0)

    plsc.subcore_barrier()
    for k in range(RPT // RB):
        r0 = s * RPT + k * RB
        pltpu.sync_copy(acc.at[pl.ds(r0, RB)], blk_v)
        pltpu.sync_copy(blk_v, out_hbm.at[pl.ds(c * HALF + r0, RB)])


@functools.partial(
    pl.kernel,
    out_type=jax.ShapeDtypeStruct((NPAD, D), jnp.float32),
    mesh=_mesh,
    scratch_types=[
        pltpu.VMEM((NCH2, K), jnp.int32),
        pltpu.VMEM((K, D), jnp.float32),
        pltpu.VMEM((RB, D), jnp.float32),
        pltpu.MemorySpace.VMEM_SHARED((HALF + 8, D), jnp.float32),
        pltpu.SemaphoreType.DMA,
    ],
)
def _deg_kernel(*refs):
    _deg_body(*refs)


def _scatter_body(h_hbm, row_hbm, col_hbm, zeros_hbm, out_hbm,
                  row_v, col_v, rows_a, rows_b, blk_v, acc,
                  gsem_a, gsem_b):
    c = lax.axis_index("c")
    s = lax.axis_index("s")
    pltpu.sync_copy(row_hbm.at[pl.ds(s * EPS, EPS)], row_v)
    pltpu.sync_copy(col_hbm.at[s], col_v)

    base = jnp.broadcast_to((c * HALF).astype(jnp.int32), (16,))
    trash = jnp.broadcast_to(jnp.int32(TRASH), (16,))

    def remap(j, _):
        for m in range(K // 16):
            v = col_v[j, pl.ds(m * 16, 16)]
            t = v - base
            ok = (t >= 0) & (t < HALF)
            col_v[j, pl.ds(m * 16, 16)] = jnp.where(ok, t, trash)
        return 0
    lax.fori_loop(0, NCH2, remap, 0)

    pltpu.sync_copy(zeros_hbm, blk_v)
    for k in range(RPT // RB):
        pltpu.sync_copy(blk_v, acc.at[pl.ds(s * RPT + k * RB, RB)])
    plsc.subcore_barrier()

    def gather(j, buf, sem):
        pltpu.async_copy(h_hbm.at[row_v.at[pl.ds(j * K, K)]], buf, sem)

    def gwait(buf, sem):
        pltpu.make_async_copy(h_hbm.at[row_v.at[pl.ds(0, K)]], buf, sem).wait()

    def scat(j, buf):
        pltpu.sync_copy(buf, acc.at[col_v.at[j]], add=True)

    gather(0, rows_a, gsem_a)

    def step(i, _):
        j = i * 2
        gwait(rows_a, gsem_a)
        gather(j + 1, rows_b, gsem_b)
        scat(j, rows_a)
        gwait(rows_b, gsem_b)
        gather(j + 2, rows_a, gsem_a)
        scat(j + 1, rows_b)
        return 0

    lax.fori_loop(0, NCH2 // 2 - 1, step, 0)
    j = NCH2 - 2
    gwait(rows_a, gsem_a)
    gather(j + 1, rows_b, gsem_b)
    scat(j, rows_a)
    gwait(rows_b, gsem_b)
    scat(j + 1, rows_b)

    plsc.subcore_barrier()
    for k in range(RPT // RB):
        r0 = s * RPT + k * RB
        pltpu.sync_copy(acc.at[pl.ds(r0, RB)], blk_v)
        pltpu.sync_copy(blk_v, out_hbm.at[pl.ds(c * HALF + r0, RB)])


@functools.partial(
    pl.kernel,
    out_type=jax.ShapeDtypeStruct((NPAD, D), jnp.float32),
    mesh=_mesh,
    scratch_types=[
        pltpu.VMEM((EPS,), jnp.int32),
        pltpu.VMEM((NCH2, K), jnp.int32),
        pltpu.VMEM((K, D), jnp.float32),
        pltpu.VMEM((K, D), jnp.float32),
        pltpu.VMEM((RB, D), jnp.float32),
        pltpu.MemorySpace.VMEM_SHARED((HALF + 8, D), jnp.float32),
        pltpu.SemaphoreType.DMA,
        pltpu.SemaphoreType.DMA,
    ],
)
def _edge_scatter(*refs):
    _scatter_body(*refs)


BS = 2000


def _mm_scale_body(x_ref, w_ref, b_ref, dis_ref, o_ref):
    acc = lax.dot_general(x_ref[...], w_ref[...], (((1,), (1,)), ((), ())),
                          preferred_element_type=jnp.float32)
    o_ref[...] = (acc + b_ref[...]) * dis_ref[...]


def _mm1(x, W, b2d, dis):
    return pl.pallas_call(
        _mm_scale_body,
        grid=(N // BS,),
        in_specs=[
            pl.BlockSpec((BS, D), lambda i: (i, 0)),
            pl.BlockSpec((D, D), lambda i: (0, 0)),
            pl.BlockSpec((1, D), lambda i: (0, 0)),
            pl.BlockSpec((BS, 1), lambda i: (i, 0)),
        ],
        out_specs=pl.BlockSpec((BS, D), lambda i: (i, 0)),
        out_shape=jax.ShapeDtypeStruct((N, D), jnp.float32),
    )(x, W, b2d, dis)


def _mid_body(sp_ref, h_ref, w_ref, b_ref, dis_ref, o_ref):
    u = jnp.maximum((sp_ref[...] + h_ref[...]) * dis_ref[...], 0.0)
    acc = lax.dot_general(u, w_ref[...], (((1,), (1,)), ((), ())),
                          preferred_element_type=jnp.float32)
    o_ref[...] = (acc + b_ref[...]) * dis_ref[...]


def _mm2(sp, h, W, b2d, dis):
    return pl.pallas_call(
        _mid_body,
        grid=(N // BS,),
        in_specs=[
            pl.BlockSpec((BS, D), lambda i: (i, 0)),
            pl.BlockSpec((BS, D), lambda i: (i, 0)),
            pl.BlockSpec((D, D), lambda i: (0, 0)),
            pl.BlockSpec((1, D), lambda i: (0, 0)),
            pl.BlockSpec((BS, 1), lambda i: (i, 0)),
        ],
        out_specs=pl.BlockSpec((BS, D), lambda i: (i, 0)),
        out_shape=jax.ShapeDtypeStruct((N, D), jnp.float32),
    )(sp, h, W, b2d, dis)


def _final_body(sp_ref, h_ref, dis_ref, o_ref):
    o_ref[...] = jnp.maximum((sp_ref[...] + h_ref[...]) * dis_ref[...], 0.0)


def _mm3(sp, h, dis):
    return pl.pallas_call(
        _final_body,
        grid=(N // BS,),
        in_specs=[
            pl.BlockSpec((BS, D), lambda i: (i, 0)),
            pl.BlockSpec((BS, D), lambda i: (i, 0)),
            pl.BlockSpec((BS, 1), lambda i: (i, 0)),
        ],
        out_specs=pl.BlockSpec((BS, D), lambda i: (i, 0)),
        out_shape=jax.ShapeDtypeStruct((N, D), jnp.float32),
    )(sp, h, dis)


def kernel(x, edge_index, W1, b1, W2, b2):
    x2 = x[0]
    row = edge_index[0].astype(jnp.int32)
    col = edge_index[1].astype(jnp.int32)
    rowd_s = row.reshape(NS, NCH2, K)
    col_s = col.reshape(NS, NCH2, K)
    zeros128 = jnp.zeros((RB, D), jnp.float32)
    ones128 = jnp.ones((K, D), jnp.float32)
    b1_2d = b1.reshape(1, D)
    b2_2d = b2.reshape(1, D)

    degw = _deg_kernel(rowd_s, ones128, zeros128)
    deg = degw[:N, 0] + 1.0
    dis = lax.rsqrt(deg)[:, None]

    h1 = _mm1(x2, W1, b1_2d, dis)
    s1 = _edge_scatter(h1, row, col_s, zeros128)
    h2 = _mm2(s1, h1, W2, b2_2d, dis)
    s2 = _edge_scatter(h2, row, col_s, zeros128)
    out = _mm3(s2, h2, dis)
    return out[None]


# speedup vs baseline: 1.3272x; 1.0001x over previous
import functools

import jax
import jax.numpy as jnp
from jax import lax
from jax.experimental import pallas as pl
from jax.experimental.pallas import tpu as pltpu
from jax.experimental.pallas import tpu_sc as plsc

N = 10000
E = 320000
D = 128
NC = 2
NS = 16
NW = NC * NS
K = 80
NCH2 = 250
EPS = NCH2 * K
E_PAD = NS * EPS
NPAD = 10240
HALF = NPAD // NC
TRASH = HALF
RPT = HALF // NS
RB = 32

_mesh = plsc.VectorSubcoreMesh(core_axis_name="c", subcore_axis_name="s")


def _deg_body(col_hbm, ones_hbm, zeros_hbm, out_hbm,
              col_v, ones_v, blk_v, acc, sem):
    c = lax.axis_index("c")
    s = lax.axis_index("s")
    pltpu.sync_copy(col_hbm.at[s], col_v)
    pltpu.sync_copy(ones_hbm, ones_v)

    base = jnp.broadcast_to((c * HALF).astype(jnp.int32), (16,))
    trash = jnp.broadcast_to(jnp.int32(TRASH), (16,))

    def remap(j, _):
        for m in range(K // 16):
            v = col_v[j, pl.ds(m * 16, 16)]
            t = v - base
            ok = (t >= 0) & (t < HALF)
            col_v[j, pl.ds(m * 16, 16)] = jnp.where(ok, t, trash)
        return 0
    lax.fori_loop(0, NCH2, remap, 0)

    pltpu.sync_copy(zeros_hbm, blk_v)
    for k in range(RPT // RB):
        pltpu.sync_copy(blk_v, acc.at[pl.ds(s * RPT + k * RB, RB)])
    plsc.subcore_barrier()

    def sstart(j):
        pltpu.async_copy(ones_v, acc.at[col_v.at[j]], sem, add=True)

    def swait():
        pltpu.make_async_copy(ones_v, acc.at[col_v.at[0]], sem).wait()

    sstart(0)
    sstart(1)

    def chunk(j, _):
        swait()
        sstart(j + 2)
        return 0
    lax.fori_loop(0, NCH2 - 2, chunk, 0)
    swait()
    swait()

    plsc.subcore_barrier()
    for k in range(RPT // RB):
        r0 = s * RPT + k * RB
        pltpu.sync_copy(acc.at[pl.ds(r0, RB)], blk_v)
        pltpu.sync_copy(blk_v, out_hbm.at[pl.ds(c * HALF + r0, RB)])


@functools.partial(
    pl.kernel,
    out_type=jax.ShapeDtypeStruct((NPAD, D), jnp.float32),
    mesh=_mesh,
    scratch_types=[
        pltpu.VMEM((NCH2, K), jnp.int32),
        pltpu.VMEM((K, D), jnp.float32),
        pltpu.VMEM((RB, D), jnp.float32),
        pltpu.MemorySpace.VMEM_SHARED((HALF + 8, D), jnp.float32),
        pltpu.SemaphoreType.DMA,
    ],
)
def _deg_kernel(*refs):
    _deg_body(*refs)


def _scatter_body(h_hbm, row_hbm, col_hbm, zeros_hbm, out_hbm,
                  row_v, col_v, rows_a, rows_b, blk_v, acc,
                  gsem_a, gsem_b):
    c = lax.axis_index("c")
    s = lax.axis_index("s")
    pltpu.sync_copy(row_hbm.at[pl.ds(s * EPS, EPS)], row_v)
    pltpu.sync_copy(col_hbm.at[s], col_v)

    base = jnp.broadcast_to((c * HALF).astype(jnp.int32), (16,))
    trash = jnp.broadcast_to(jnp.int32(TRASH), (16,))

    def remap(j, _):
        for m in range(K // 16):
            v = col_v[j, pl.ds(m * 16, 16)]
            t = v - base
            ok = (t >= 0) & (t < HALF)
            col_v[j, pl.ds(m * 16, 16)] = jnp.where(ok, t, trash)
        return 0
    lax.fori_loop(0, NCH2, remap, 0)

    pltpu.sync_copy(zeros_hbm, blk_v)
    for k in range(RPT // RB):
        pltpu.sync_copy(blk_v, acc.at[pl.ds(s * RPT + k * RB, RB)])
    plsc.subcore_barrier()

    def gather(j, buf, sem):
        pltpu.async_copy(h_hbm.at[row_v.at[pl.ds(j * K, K)]], buf, sem)

    def gwait(buf, sem):
        pltpu.make_async_copy(h_hbm.at[row_v.at[pl.ds(0, K)]], buf, sem).wait()

    def scat(j, buf):
        pltpu.sync_copy(buf, acc.at[col_v.at[j]], add=True)

    gather(0, rows_a, gsem_a)

    def step(i, _):
        j = i * 2
        gwait(rows_a, gsem_a)
        gather(j + 1, rows_b, gsem_b)
        scat(j, rows_a)
        gwait(rows_b, gsem_b)
        gather(j + 2, rows_a, gsem_a)
        scat(j + 1, rows_b)
        return 0

    lax.fori_loop(0, NCH2 // 2 - 1, step, 0)
    j = NCH2 - 2
    gwait(rows_a, gsem_a)
    gather(j + 1, rows_b, gsem_b)
    scat(j, rows_a)
    gwait(rows_b, gsem_b)
    scat(j + 1, rows_b)

    plsc.subcore_barrier()
    for k in range(RPT // RB):
        r0 = s * RPT + k * RB
        pltpu.sync_copy(acc.at[pl.ds(r0, RB)], blk_v)
        pltpu.sync_copy(blk_v, out_hbm.at[pl.ds(c * HALF + r0, RB)])


@functools.partial(
    pl.kernel,
    out_type=jax.ShapeDtypeStruct((NPAD, D), jnp.float32),
    mesh=_mesh,
    scratch_types=[
        pltpu.VMEM((EPS,), jnp.int32),
        pltpu.VMEM((NCH2, K), jnp.int32),
        pltpu.VMEM((K, D), jnp.float32),
        pltpu.VMEM((K, D), jnp.float32),
        pltpu.VMEM((RB, D), jnp.float32),
        pltpu.MemorySpace.VMEM_SHARED((HALF + 8, D), jnp.float32),
        pltpu.SemaphoreType.DMA,
        pltpu.SemaphoreType.DMA,
    ],
)
def _edge_scatter(*refs):
    _scatter_body(*refs)


BS = 2000


def _mm_scale_body(x_ref, w_ref, b_ref, dis_ref, o_ref):
    acc = lax.dot_general(x_ref[...], w_ref[...], (((1,), (1,)), ((), ())),
                          preferred_element_type=jnp.float32)
    o_ref[...] = (acc + b_ref[...]) * dis_ref[...]


def _mm1(x, W, b2d, dis):
    return pl.pallas_call(
        _mm_scale_body,
        grid=(N // BS,),
        in_specs=[
            pl.BlockSpec((BS, D), lambda i: (i, 0)),
            pl.BlockSpec((D, D), lambda i: (0, 0)),
            pl.BlockSpec((1, D), lambda i: (0, 0)),
            pl.BlockSpec((BS, 1), lambda i: (i, 0)),
        ],
        out_specs=pl.BlockSpec((BS, D), lambda i: (i, 0)),
        out_shape=jax.ShapeDtypeStruct((N, D), jnp.float32),
    )(x, W, b2d, dis)


def _mid_body(sp_ref, h_ref, w_ref, b_ref, dis_ref, o_ref):
    u = jnp.maximum((sp_ref[...] + h_ref[...]) * dis_ref[...], 0.0)
    acc = lax.dot_general(u, w_ref[...], (((1,), (1,)), ((), ())),
                          preferred_element_type=jnp.float32)
    o_ref[...] = (acc + b_ref[...]) * dis_ref[...]


def _mm2(sp, h, W, b2d, dis):
    return pl.pallas_call(
        _mid_body,
        grid=(N // BS,),
        in_specs=[
            pl.BlockSpec((BS, D), lambda i: (i, 0)),
            pl.BlockSpec((BS, D), lambda i: (i, 0)),
            pl.BlockSpec((D, D), lambda i: (0, 0)),
            pl.BlockSpec((1, D), lambda i: (0, 0)),
            pl.BlockSpec((BS, 1), lambda i: (i, 0)),
        ],
        out_specs=pl.BlockSpec((BS, D), lambda i: (i, 0)),
        out_shape=jax.ShapeDtypeStruct((N, D), jnp.float32),
    )(sp, h, W, b2d, dis)


def _final_body(sp_ref, h_ref, dis_ref, o_ref):
    o_ref[...] = jnp.maximum((sp_ref[...] + h_ref[...]) * dis_ref[...], 0.0)


def _mm3(sp, h, dis):
    return pl.pallas_call(
        _final_body,
        grid=(N // BS,),
        in_specs=[
            pl.BlockSpec((BS, D), lambda i: (i, 0)),
            pl.BlockSpec((BS, D), lambda i: (i, 0)),
            pl.BlockSpec((BS, 1), lambda i: (i, 0)),
        ],
        out_specs=pl.BlockSpec((BS, D), lambda i: (i, 0)),
        out_shape=jax.ShapeDtypeStruct((N, D), jnp.float32),
    )(sp, h, dis)


def kernel(x, edge_index, W1, b1, W2, b2):
    x2 = x[0]
    row = edge_index[0].astype(jnp.int32)
    col = edge_index[1].astype(jnp.int32)
    rowd_s = row.reshape(NS, NCH2, K)
    col_s = col.reshape(NS, NCH2, K)
    zeros128 = jnp.zeros((RB, D), jnp.float32)
    ones128 = jnp.ones((K, D), jnp.float32)
    b1_2d = b1.reshape(1, D)
    b2_2d = b2.reshape(1, D)

    degw = _deg_kernel(rowd_s, ones128, zeros128)
    deg = degw[:N, 0] + 1.0
    dis = lax.rsqrt(deg)[:, None]

    h1 = _mm1(x2, W1, b1_2d, dis)
    s1 = _edge_scatter(h1, row, col_s, zeros128)
    h2 = _mm2(s1, h1, W2, b2_2d, dis)
    s2 = _edge_scatter(h2, row, col_s, zeros128)
    out = _mm3(s2, h2, dis)
    return out[None]


# speedup vs baseline: 1.3291x; 1.0014x over previous
import functools

import jax
import jax.numpy as jnp
from jax import lax
from jax.experimental import pallas as pl
from jax.experimental.pallas import tpu as pltpu
from jax.experimental.pallas import tpu_sc as plsc

N = 10000
E = 320000
D = 128
NC = 2
NS = 16
NW = NC * NS
K = 80
NCH2 = 250
EPS = NCH2 * K
E_PAD = NS * EPS
NPAD = 10240
HALF = NPAD // NC
TRASH = HALF
RPT = HALF // NS
RB = 32

_mesh = plsc.VectorSubcoreMesh(core_axis_name="c", subcore_axis_name="s")


def _deg_body(col_hbm, ones_hbm, zeros_hbm, out_hbm,
              col_v, ones_v, blk_v, acc, sem):
    c = lax.axis_index("c")
    s = lax.axis_index("s")
    pltpu.sync_copy(col_hbm.at[s], col_v)
    pltpu.sync_copy(ones_hbm, ones_v)

    base = jnp.broadcast_to((c * HALF).astype(jnp.int32), (16,))
    trash = jnp.broadcast_to(jnp.int32(TRASH), (16,))

    def remap(j, _):
        for m in range(K // 16):
            v = col_v[j, pl.ds(m * 16, 16)]
            t = v - base
            ok = (t >= 0) & (t < HALF)
            col_v[j, pl.ds(m * 16, 16)] = jnp.where(ok, t, trash)
        return 0
    lax.fori_loop(0, NCH2, remap, 0)

    pltpu.sync_copy(zeros_hbm, blk_v)
    for k in range(RPT // RB):
        pltpu.sync_copy(blk_v, acc.at[pl.ds(s * RPT + k * RB, RB)])
    plsc.subcore_barrier()

    def chunk(j, _):
        pltpu.sync_copy(ones_v, acc.at[col_v.at[j]], add=True)
        return 0
    lax.fori_loop(0, NCH2, chunk, 0)

    plsc.subcore_barrier()
    for k in range(RPT // RB):
        r0 = s * RPT + k * RB
        pltpu.sync_copy(acc.at[pl.ds(r0, RB)], blk_v)
        pltpu.sync_copy(blk_v, out_hbm.at[pl.ds(c * HALF + r0, RB)])


@functools.partial(
    pl.kernel,
    out_type=jax.ShapeDtypeStruct((NPAD, D), jnp.float32),
    mesh=_mesh,
    scratch_types=[
        pltpu.VMEM((NCH2, K), jnp.int32),
        pltpu.VMEM((K, D), jnp.float32),
        pltpu.VMEM((RB, D), jnp.float32),
        pltpu.MemorySpace.VMEM_SHARED((HALF + 8, D), jnp.float32),
        pltpu.SemaphoreType.DMA,
    ],
)
def _deg_kernel(*refs):
    _deg_body(*refs)


def _scatter_body(h_hbm, row_hbm, col_hbm, zeros_hbm, out_hbm,
                  row_v, col_v, rows_a, rows_b, blk_v, acc,
                  gsem_a, gsem_b):
    c = lax.axis_index("c")
    s = lax.axis_index("s")
    pltpu.sync_copy(row_hbm.at[pl.ds(s * EPS, EPS)], row_v)
    pltpu.sync_copy(col_hbm.at[s], col_v)

    base = jnp.broadcast_to((c * HALF).astype(jnp.int32), (16,))
    trash = jnp.broadcast_to(jnp.int32(TRASH), (16,))

    def remap(j, _):
        for m in range(K // 16):
            v = col_v[j, pl.ds(m * 16, 16)]
            t = v - base
            ok = (t >= 0) & (t < HALF)
            col_v[j, pl.ds(m * 16, 16)] = jnp.where(ok, t, trash)
        return 0
    lax.fori_loop(0, NCH2, remap, 0)

    pltpu.sync_copy(zeros_hbm, blk_v)
    for k in range(RPT // RB):
        pltpu.sync_copy(blk_v, acc.at[pl.ds(s * RPT + k * RB, RB)])
    plsc.subcore_barrier()

    def gather(j, buf, sem):
        pltpu.async_copy(h_hbm.at[row_v.at[pl.ds(j * K, K)]], buf, sem)

    def gwait(buf, sem):
        pltpu.make_async_copy(h_hbm.at[row_v.at[pl.ds(0, K)]], buf, sem).wait()

    def scat(j, buf):
        pltpu.sync_copy(buf, acc.at[col_v.at[j]], add=True)

    gather(0, rows_a, gsem_a)

    def step(i, _):
        j = i * 2
        gwait(rows_a, gsem_a)
        gather(j + 1, rows_b, gsem_b)
        scat(j, rows_a)
        gwait(rows_b, gsem_b)
        gather(j + 2, rows_a, gsem_a)
        scat(j + 1, rows_b)
        return 0

    lax.fori_loop(0, NCH2 // 2 - 1, step, 0)
    j = NCH2 - 2
    gwait(rows_a, gsem_a)
    gather(j + 1, rows_b, gsem_b)
    scat(j, rows_a)
    gwait(rows_b, gsem_b)
    scat(j + 1, rows_b)

    plsc.subcore_barrier()
    for k in range(RPT // RB):
        r0 = s * RPT + k * RB
        pltpu.sync_copy(acc.at[pl.ds(r0, RB)], blk_v)
        pltpu.sync_copy(blk_v, out_hbm.at[pl.ds(c * HALF + r0, RB)])


@functools.partial(
    pl.kernel,
    out_type=jax.ShapeDtypeStruct((NPAD, D), jnp.float32),
    mesh=_mesh,
    scratch_types=[
        pltpu.VMEM((EPS,), jnp.int32),
        pltpu.VMEM((NCH2, K), jnp.int32),
        pltpu.VMEM((K, D), jnp.float32),
        pltpu.VMEM((K, D), jnp.float32),
        pltpu.VMEM((RB, D), jnp.float32),
        pltpu.MemorySpace.VMEM_SHARED((HALF + 8, D), jnp.float32),
        pltpu.SemaphoreType.DMA,
        pltpu.SemaphoreType.DMA,
    ],
)
def _edge_scatter(*refs):
    _scatter_body(*refs)


BS = 2000


def _mm_scale_body(x_ref, w_ref, b_ref, dis_ref, o_ref):
    acc = lax.dot_general(x_ref[...], w_ref[...], (((1,), (1,)), ((), ())),
                          preferred_element_type=jnp.float32)
    o_ref[...] = (acc + b_ref[...]) * dis_ref[...]


def _mm1(x, W, b2d, dis):
    return pl.pallas_call(
        _mm_scale_body,
        grid=(N // BS,),
        in_specs=[
            pl.BlockSpec((BS, D), lambda i: (i, 0)),
            pl.BlockSpec((D, D), lambda i: (0, 0)),
            pl.BlockSpec((1, D), lambda i: (0, 0)),
            pl.BlockSpec((BS, 1), lambda i: (i, 0)),
        ],
        out_specs=pl.BlockSpec((BS, D), lambda i: (i, 0)),
        out_shape=jax.ShapeDtypeStruct((N, D), jnp.float32),
    )(x, W, b2d, dis)


def _mid_body(sp_ref, h_ref, w_ref, b_ref, dis_ref, o_ref):
    u = jnp.maximum((sp_ref[...] + h_ref[...]) * dis_ref[...], 0.0)
    acc = lax.dot_general(u, w_ref[...], (((1,), (1,)), ((), ())),
                          preferred_element_type=jnp.float32)
    o_ref[...] = (acc + b_ref[...]) * dis_ref[...]


def _mm2(sp, h, W, b2d, dis):
    return pl.pallas_call(
        _mid_body,
        grid=(N // BS,),
        in_specs=[
            pl.BlockSpec((BS, D), lambda i: (i, 0)),
            pl.BlockSpec((BS, D), lambda i: (i, 0)),
            pl.BlockSpec((D, D), lambda i: (0, 0)),
            pl.BlockSpec((1, D), lambda i: (0, 0)),
            pl.BlockSpec((BS, 1), lambda i: (i, 0)),
        ],
        out_specs=pl.BlockSpec((BS, D), lambda i: (i, 0)),
        out_shape=jax.ShapeDtypeStruct((N, D), jnp.float32),
    )(sp, h, W, b2d, dis)


def _final_body(sp_ref, h_ref, dis_ref, o_ref):
    o_ref[...] = jnp.maximum((sp_ref[...] + h_ref[...]) * dis_ref[...], 0.0)


def _mm3(sp, h, dis):
    return pl.pallas_call(
        _final_body,
        grid=(N // BS,),
        in_specs=[
            pl.BlockSpec((BS, D), lambda i: (i, 0)),
            pl.BlockSpec((BS, D), lambda i: (i, 0)),
            pl.BlockSpec((BS, 1), lambda i: (i, 0)),
        ],
        out_specs=pl.BlockSpec((BS, D), lambda i: (i, 0)),
        out_shape=jax.ShapeDtypeStruct((N, D), jnp.float32),
    )(sp, h, dis)


def kernel(x, edge_index, W1, b1, W2, b2):
    x2 = x[0]
    row = edge_index[0].astype(jnp.int32)
    col = edge_index[1].astype(jnp.int32)
    rowd_s = row.reshape(NS, NCH2, K)
    col_s = col.reshape(NS, NCH2, K)
    zeros128 = jnp.zeros((RB, D), jnp.float32)
    ones128 = jnp.ones((K, D), jnp.float32)
    b1_2d = b1.reshape(1, D)
    b2_2d = b2.reshape(1, D)

    degw = _deg_kernel(rowd_s, ones128, zeros128)
    deg = degw[:N, 0] + 1.0
    dis = lax.rsqrt(deg)[:, None]

    h1 = _mm1(x2, W1, b1_2d, dis)
    s1 = _edge_scatter(h1, row, col_s, zeros128)
    h2 = _mm2(s1, h1, W2, b2_2d, dis)
    s2 = _edge_scatter(h2, row, col_s, zeros128)
    out = _mm3(s2, h2, dis)
    return out[None]
